# Initial kernel scaffold; baseline (speedup 1.0000x reference)
import jax
import jax.numpy as jnp
from jax.experimental import pallas as pl


def kernel(x, params):
    raise NotImplementedError("write your pallas kernel here")


# speedup vs baseline: 6.9122x; 6.9122x over previous
import functools

import jax
import jax.numpy as jnp
from jax.experimental import pallas as pl
from jax.experimental.pallas import tpu as pltpu
from jax.experimental.pallas import tpu_sc as plsc

B, N, K = 4, 4096, 20
C0 = 9
RB = 256
NBLK = N // RB
NB = 512
FB = N // NB
KPAD = 32
NEG = -3.0e38
EPS = 1e-5
CNT_EDGE = float(B * N * K)
CNT_PT = float(B * N)
F32 = jnp.float32
BF16 = jnp.bfloat16


def _lrelu(v):
    return jnp.where(v > 0, v, 0.2 * v)


def _bdot(a, b_ref):
    return jnp.dot(a.astype(BF16), b_ref[...], preferred_element_type=F32)


def _affine(st_ref, g_ref, b_ref, cnt):
    s = st_ref[0:1, :]
    q = st_ref[1:2, :]
    mu = s / cnt
    var = q / cnt - mu * mu
    inv = jax.lax.rsqrt(var + EPS)
    scale = g_ref[...] * inv
    shift = b_ref[...] - mu * scale
    return scale, shift


def _topk_loop(b, d_ref, idx_ref):
    iota_n = jax.lax.broadcasted_iota(jnp.int32, (RB, N), 1)
    iota_k = jax.lax.broadcasted_iota(jnp.int32, (RB, KPAD), 1)
    base = b * N

    def step(j, acc):
        d = d_ref[...]
        m = jnp.max(d, axis=1, keepdims=True)
        sel = jnp.where(d >= m, iota_n, N)
        idxv = jnp.min(sel, axis=1, keepdims=True)
        d_ref[...] = jnp.where(iota_n == idxv, NEG, d)
        return jnp.where(iota_k == j, idxv + base, acc)

    acc0 = jnp.zeros((RB, KPAD), jnp.int32)
    idx_ref[...] = jax.lax.fori_loop(0, K, step, acc0)


def _xx_rows(xr, c):
    sq = xr * xr
    if c == C0:
        acc = sq[:, 0:1]
        for j in range(1, c):
            acc = acc + sq[:, j:j + 1]
        return acc
    acc = sq[:, 0:8]
    for t in range(1, c // 8):
        acc = acc + sq[:, 8 * t:8 * t + 8]
    for off in (4, 2, 1):
        acc = acc[:, :off] + acc[:, off:2 * off]
    return acc


def _xx_cols(xc, c):
    sq = xc * xc
    if c == C0:
        acc = sq[0:1]
        for j in range(1, c):
            acc = acc + sq[j:j + 1]
        return acc
    acc = sq[0:8]
    for t in range(1, c // 8):
        acc = acc + sq[8 * t:8 * t + 8]
    for off in (4, 2, 1):
        acc = acc[:off] + acc[off:2 * off]
    return acc


def _knn_body(xr_ref, xc_ref, idx_ref, d_ref):
    b = pl.program_id(0)
    xr = xr_ref[...]
    xc = xc_ref[0]
    c = xr.shape[1]
    d = 2.0 * jnp.dot(xr.astype(BF16), xc.astype(BF16),
                      preferred_element_type=F32)
    d_ref[...] = d - _xx_rows(xr, c) - _xx_cols(xc, c)
    _topk_loop(b, d_ref, idx_ref)


def _knn(x_rows, x_cols):
    c = x_rows.shape[1]
    return pl.pallas_call(
        _knn_body,
        grid=(B, NBLK),
        in_specs=[
            pl.BlockSpec((RB, c), lambda b, i: (b * NBLK + i, 0)),
            pl.BlockSpec((1, c, N), lambda b, i: (b, 0, 0)),
        ],
        out_specs=pl.BlockSpec((RB, KPAD), lambda b, i: (b * NBLK + i, 0)),
        out_shape=jax.ShapeDtypeStruct((B * N, KPAD), jnp.int32),
        scratch_shapes=[pltpu.VMEM((RB, N), F32)],
    )(x_rows, x_cols)


def _prep_idx(idxp):
    return idxp.reshape(B, N, KPAD)[:, :, :K].transpose(0, 2, 1).reshape(
        B * K * N)


_GW = 128
_NWORK = 32


@jax.jit
def _sc_gather(table, idxg):
    m = idxg.shape[0]
    per_w = m // _NWORK
    nch = per_w // _GW
    mesh = plsc.VectorSubcoreMesh(core_axis_name="c", subcore_axis_name="s")

    @functools.partial(
        pl.kernel,
        out_type=jax.ShapeDtypeStruct((m, 128), table.dtype),
        mesh=mesh,
        scratch_types=[
            pltpu.VMEM((_GW,), jnp.int32),
            pltpu.VMEM((_GW, 128), F32),
            pltpu.SemaphoreType.DMA,
        ])
    def kern(tab_hbm, i_hbm, o_hbm, idx_v, rows_v, sem):
        wid = jax.lax.axis_index("s") * 2 + jax.lax.axis_index("c")
        base = wid * per_w

        @pl.loop(0, nch)
        def _(i):
            off = base + i * _GW
            pltpu.sync_copy(i_hbm.at[pl.ds(off, _GW)], idx_v)
            pltpu.async_copy(tab_hbm.at[idx_v], rows_v, sem).wait()
            pltpu.sync_copy(rows_v, o_hbm.at[pl.ds(off, _GW)])

    return kern(table, idxg)


def _feat(g, xc, c):
    return jnp.concatenate([g[:, :c] - xc, xc], axis=1).astype(BF16)


def _kadd(o_ref, val):
    s = o_ref[0:1, :]
    c = o_ref[1:2, :]
    y = val - c
    t = s + y
    o_ref[1:2, :] = (t - s) - y
    o_ref[0:1, :] = t


def _edge_norm(ssum_ref, svar_ref, bg_ref, bb_ref):
    mu = (ssum_ref[0:1, :] + ssum_ref[1:2, :]) / CNT_EDGE
    var = (svar_ref[0:1, :] + svar_ref[1:2, :]) / CNT_EDGE
    sd = jnp.sqrt(var + EPS)
    return mu, sd, bg_ref[...], bb_ref[...]


def _tsum_body(g_ref, xc_ref, w1t_ref, o_ref, *, c):
    b, k, i = pl.program_id(0), pl.program_id(1), pl.program_id(2)
    t = jnp.dot(_feat(g_ref[...], xc_ref[...], c), w1t_ref[...],
                preferred_element_type=F32)

    @pl.when((b == 0) & (k == 0) & (i == 0))
    def _():
        o_ref[...] = jnp.zeros_like(o_ref)

    _kadd(o_ref, jnp.sum(t, axis=0, keepdims=True))


def _tvar_body(g_ref, xc_ref, w1t_ref, ss_ref, o_ref, *, c):
    b, k, i = pl.program_id(0), pl.program_id(1), pl.program_id(2)
    t = jnp.dot(_feat(g_ref[...], xc_ref[...], c), w1t_ref[...],
                preferred_element_type=F32)
    mu = (ss_ref[0:1, :] + ss_ref[1:2, :]) / CNT_EDGE
    d = t - mu

    @pl.when((b == 0) & (k == 0) & (i == 0))
    def _():
        o_ref[...] = jnp.zeros_like(o_ref)

    _kadd(o_ref, jnp.sum(d * d, axis=0, keepdims=True))


def _tpass(kind, g, xrow, w1t, ssum=None):
    c = xrow.shape[1]
    specs = [
        pl.BlockSpec((NB, 128), lambda b, k, i: ((b * K + k) * FB + i, 0)),
        pl.BlockSpec((NB, c), lambda b, k, i: (b * FB + i, 0)),
        pl.BlockSpec((2 * c, 64), lambda b, k, i: (0, 0)),
    ]
    args = [g, xrow, w1t]
    if kind == 'sum':
        body = functools.partial(_tsum_body, c=c)
    else:
        body = functools.partial(_tvar_body, c=c)
        specs.append(pl.BlockSpec((8, 64), lambda b, k, i: (0, 0)))
        args.append(ssum)
    return pl.pallas_call(
        body,
        grid=(B, K, FB),
        in_specs=specs,
        out_specs=pl.BlockSpec((8, 64), lambda b, k, i: (0, 0)),
        out_shape=jax.ShapeDtypeStruct((8, 64), F32),
    )(*args)


def _convmax_body(g_ref, xc_ref, w1t_ref, ss_ref, sv_ref, bg_ref, bb_ref,
                  w2t_ref, m_ref, u_ref, st2_ref, *, c):
    b, i = pl.program_id(0), pl.program_id(1)
    mu, sd, bg, bb = _edge_norm(ss_ref, sv_ref, bg_ref, bb_ref)
    xc = xc_ref[...]
    w1t = w1t_ref[...]
    w2t = w2t_ref[...]

    @pl.when((b == 0) & (i == 0))
    def _():
        st2_ref[...] = jnp.zeros_like(st2_ref)

    def step(k, carry):
        m, s2, c2 = carry
        g = g_ref[pl.ds(k, 1), :, :][0]
        t = jnp.dot(_feat(g, xc, c), w1t, preferred_element_type=F32)
        h = _lrelu(((t - mu) / sd) * bg + bb)
        u = jnp.dot(h.astype(BF16), w2t, preferred_element_type=F32)
        u_ref[pl.ds(k, 1), :, :] = u[None]
        y = jnp.sum(u, axis=0, keepdims=True) - c2
        s2n = s2 + y
        c2 = (s2n - s2) - y
        return jnp.maximum(m, u), s2n, c2

    m0 = jnp.full((NB, 64), NEG, F32)
    z = jnp.zeros((1, 64), F32)
    m, s2, c2 = jax.lax.fori_loop(0, K, step, (m0, z, z))
    m_ref[...] = m
    _kadd(st2_ref, s2 + c2)


def _convmax(g3, xrow, w1t, ssum, svar, bg, bb, w2t):
    c = xrow.shape[1]
    body = functools.partial(_convmax_body, c=c)
    return pl.pallas_call(
        body,
        grid=(B, FB),
        in_specs=[
            pl.BlockSpec((K, NB, 128), lambda b, i: (b, i, 0)),
            pl.BlockSpec((NB, c), lambda b, i: (b * FB + i, 0)),
            pl.BlockSpec((2 * c, 64), lambda b, i: (0, 0)),
            pl.BlockSpec((8, 64), lambda b, i: (0, 0)),
            pl.BlockSpec((8, 64), lambda b, i: (0, 0)),
            pl.BlockSpec((1, 64), lambda b, i: (0, 0)),
            pl.BlockSpec((1, 64), lambda b, i: (0, 0)),
            pl.BlockSpec((64, 64), lambda b, i: (0, 0)),
        ],
        out_specs=[
            pl.BlockSpec((NB, 64), lambda b, i: (b * FB + i, 0)),
            pl.BlockSpec((K, NB, 64), lambda b, i: (b, i, 0)),
            pl.BlockSpec((8, 64), lambda b, i: (0, 0)),
        ],
        out_shape=[
            jax.ShapeDtypeStruct((B * N, 64), F32),
            jax.ShapeDtypeStruct((B * K, N, 64), F32),
            jax.ShapeDtypeStruct((8, 64), F32),
        ],
    )(g3, xrow, w1t, ssum, svar, bg, bb, w2t)


def _uvar_body(u_ref, ss_ref, o_ref):
    b, k, i = pl.program_id(0), pl.program_id(1), pl.program_id(2)
    mu = (ss_ref[0:1, :] + ss_ref[1:2, :]) / CNT_EDGE
    d = u_ref[...] - mu

    @pl.when((b == 0) & (k == 0) & (i == 0))
    def _():
        o_ref[...] = jnp.zeros_like(o_ref)

    _kadd(o_ref, jnp.sum(d * d, axis=0, keepdims=True))


def _uvar(u, ssum):
    return pl.pallas_call(
        _uvar_body,
        grid=(B, K, FB),
        in_specs=[
            pl.BlockSpec((NB, 64), lambda b, k, i: ((b * K + k) * FB + i, 0)),
            pl.BlockSpec((8, 64), lambda b, k, i: (0, 0)),
        ],
        out_specs=pl.BlockSpec((8, 64), lambda b, k, i: (0, 0)),
        out_shape=jax.ShapeDtypeStruct((8, 64), F32),
    )(u, ssum)


def _conv5_body(g_ref, xc_ref, w1t_ref, m_ref, st_ref, *, c):
    b, i = pl.program_id(0), pl.program_id(1)
    xc = xc_ref[...]
    w1t = w1t_ref[...]

    @pl.when((b == 0) & (i == 0))
    def _():
        st_ref[...] = jnp.zeros_like(st_ref)

    def step(k, carry):
        m, s, q = carry
        g = g_ref[pl.ds(k, 1), :, :][0]
        t = jnp.dot(_feat(g, xc, c), w1t, preferred_element_type=F32)
        s = s + jnp.sum(t, axis=0, keepdims=True)
        q = q + jnp.sum(t * t, axis=0, keepdims=True)
        return jnp.maximum(m, t), s, q

    m0 = jnp.full((NB, 64), NEG, F32)
    z = jnp.zeros((1, 64), F32)
    m, s, q = jax.lax.fori_loop(0, K, step, (m0, z, z))
    m_ref[...] = m
    st_ref[0:1, :] += s
    st_ref[1:2, :] += q


def _conv5(g3, xrow, w1t):
    c = xrow.shape[1]
    body = functools.partial(_conv5_body, c=c)
    return pl.pallas_call(
        body,
        grid=(B, FB),
        in_specs=[
            pl.BlockSpec((K, NB, 128), lambda b, i: (b, i, 0)),
            pl.BlockSpec((NB, c), lambda b, i: (b * FB + i, 0)),
            pl.BlockSpec((2 * c, 64), lambda b, i: (0, 0)),
        ],
        out_specs=[
            pl.BlockSpec((NB, 64), lambda b, i: (b * FB + i, 0)),
            pl.BlockSpec((8, 64), lambda b, i: (0, 0)),
        ],
        out_shape=[
            jax.ShapeDtypeStruct((B * N, 64), F32),
            jax.ShapeDtypeStruct((8, 64), F32),
        ],
    )(g3, xrow, w1t)


def _fin_body(m_ref, ss_ref, sv_ref, bg_ref, bb_ref, x_ref, xp_ref):
    mu, sd, bg, bb = _edge_norm(ss_ref, sv_ref, bg_ref, bb_ref)
    x = _lrelu(((m_ref[...] - mu) / sd) * bg + bb)
    x_ref[...] = x
    xp_ref[...] = jnp.concatenate([x, jnp.zeros((NB, 64), F32)], axis=1)


def _fin(m, ssum, svar, bg, bb):
    return pl.pallas_call(
        _fin_body,
        grid=(B * FB,),
        in_specs=[
            pl.BlockSpec((NB, 64), lambda i: (i, 0)),
            pl.BlockSpec((8, 64), lambda i: (0, 0)),
            pl.BlockSpec((8, 64), lambda i: (0, 0)),
            pl.BlockSpec((1, 64), lambda i: (0, 0)),
            pl.BlockSpec((1, 64), lambda i: (0, 0)),
        ],
        out_specs=[
            pl.BlockSpec((NB, 64), lambda i: (i, 0)),
            pl.BlockSpec((NB, 128), lambda i: (i, 0)),
        ],
        out_shape=[
            jax.ShapeDtypeStruct((B * N, 64), F32),
            jax.ShapeDtypeStruct((B * N, 128), F32),
        ],
    )(m, ssum, svar, bg, bb)


def _fin3_body(m_ref, st_ref, bg_ref, bb_ref, x_ref):
    scale, shift = _affine(st_ref, bg_ref, bb_ref, CNT_EDGE)
    x_ref[...] = _lrelu(m_ref[...] * scale + shift)


def _fin3(m, st, bg, bb):
    return pl.pallas_call(
        _fin3_body,
        grid=(B * FB,),
        in_specs=[
            pl.BlockSpec((NB, 64), lambda i: (i, 0)),
            pl.BlockSpec((8, 64), lambda i: (0, 0)),
            pl.BlockSpec((1, 64), lambda i: (0, 0)),
            pl.BlockSpec((1, 64), lambda i: (0, 0)),
        ],
        out_specs=pl.BlockSpec((NB, 64), lambda i: (i, 0)),
        out_shape=jax.ShapeDtypeStruct((B * N, 64), F32),
    )(m, st, bg, bb)


def _head1_body(x1_ref, x2_ref, x3_ref, wa_ref, wb_ref, wc_ref,
                st_ref, gm_ref):
    b, i = pl.program_id(0), pl.program_id(1)
    u = (_bdot(x1_ref[...], wa_ref)
         + _bdot(x2_ref[...], wb_ref)
         + _bdot(x3_ref[...], wc_ref))

    @pl.when((b == 0) & (i == 0))
    def _():
        st_ref[...] = jnp.zeros_like(st_ref)

    @pl.when(i == 0)
    def _():
        gm_ref[...] = jnp.full_like(gm_ref, NEG)

    st_ref[0:1, :] += jnp.sum(u, axis=0, keepdims=True)
    st_ref[1:2, :] += jnp.sum(u * u, axis=0, keepdims=True)
    gm_ref[0] = jnp.maximum(gm_ref[0], jnp.max(u, axis=0, keepdims=True))


def _head1(x1, x2, x3, wa, wb, wc):
    return pl.pallas_call(
        _head1_body,
        grid=(B, FB),
        in_specs=[
            pl.BlockSpec((NB, 64), lambda b, i: (b * FB + i, 0)),
            pl.BlockSpec((NB, 64), lambda b, i: (b * FB + i, 0)),
            pl.BlockSpec((NB, 64), lambda b, i: (b * FB + i, 0)),
            pl.BlockSpec((64, 1024), lambda b, i: (0, 0)),
            pl.BlockSpec((64, 1024), lambda b, i: (0, 0)),
            pl.BlockSpec((64, 1024), lambda b, i: (0, 0)),
        ],
        out_specs=[
            pl.BlockSpec((8, 1024), lambda b, i: (0, 0)),
            pl.BlockSpec((1, 1, 1024), lambda b, i: (b, 0, 0)),
        ],
        out_shape=[
            jax.ShapeDtypeStruct((8, 1024), F32),
            jax.ShapeDtypeStruct((B, 1, 1024), F32),
        ],
    )(x1, x2, x3, wa, wb, wc)


def _head2_body(gm_ref, st_ref, bg_ref, bb_ref, w_ref, c_ref):
    scale, shift = _affine(st_ref, bg_ref, bb_ref, CNT_PT)
    e = _lrelu(gm_ref[:, 0, :] * scale + shift)
    c_ref[:, 0, :] = _bdot(e, w_ref)


def _head2(gm, st, bg, bb, w7g):
    return pl.pallas_call(
        _head2_body,
        grid=(1,),
        in_specs=[
            pl.BlockSpec((B, 1, 1024), lambda i: (0, 0, 0)),
            pl.BlockSpec((8, 1024), lambda i: (0, 0)),
            pl.BlockSpec((1, 1024), lambda i: (0, 0)),
            pl.BlockSpec((1, 1024), lambda i: (0, 0)),
            pl.BlockSpec((1024, 512), lambda i: (0, 0)),
        ],
        out_specs=pl.BlockSpec((B, 1, 512), lambda i: (0, 0, 0)),
        out_shape=jax.ShapeDtypeStruct((B, 1, 512), F32),
    )(gm, st, bg, bb, w7g)


def _head3_body(x1_ref, x2_ref, x3_ref, w1_ref, w2_ref, w3_ref, c7_ref,
                u_ref, st_ref):
    b, i = pl.program_id(0), pl.program_id(1)
    u = (_bdot(x1_ref[...], w1_ref)
         + _bdot(x2_ref[...], w2_ref)
         + _bdot(x3_ref[...], w3_ref)
         + c7_ref[0])

    @pl.when((b == 0) & (i == 0))
    def _():
        st_ref[...] = jnp.zeros_like(st_ref)

    u_ref[...] = u
    st_ref[0:1, :] += jnp.sum(u, axis=0, keepdims=True)
    st_ref[1:2, :] += jnp.sum(u * u, axis=0, keepdims=True)


def _head3(x1, x2, x3, w1, w2, w3, c7):
    return pl.pallas_call(
        _head3_body,
        grid=(B, FB),
        in_specs=[
            pl.BlockSpec((NB, 64), lambda b, i: (b * FB + i, 0)),
            pl.BlockSpec((NB, 64), lambda b, i: (b * FB + i, 0)),
            pl.BlockSpec((NB, 64), lambda b, i: (b * FB + i, 0)),
            pl.BlockSpec((64, 512), lambda b, i: (0, 0)),
            pl.BlockSpec((64, 512), lambda b, i: (0, 0)),
            pl.BlockSpec((64, 512), lambda b, i: (0, 0)),
            pl.BlockSpec((1, 1, 512), lambda b, i: (b, 0, 0)),
        ],
        out_specs=[
            pl.BlockSpec((NB, 512), lambda b, i: (b * FB + i, 0)),
            pl.BlockSpec((8, 512), lambda b, i: (0, 0)),
        ],
        out_shape=[
            jax.ShapeDtypeStruct((B * N, 512), F32),
            jax.ShapeDtypeStruct((8, 512), F32),
        ],
    )(x1, x2, x3, w1, w2, w3, c7)


def _mlp_body(u_ref, st_ref, bg_ref, bb_ref, w_ref, o_ref, so_ref, *,
              with_stats):
    i = pl.program_id(0)
    scale, shift = _affine(st_ref, bg_ref, bb_ref, CNT_PT)
    r = _lrelu(u_ref[...] * scale + shift)
    o = _bdot(r, w_ref)
    o_ref[...] = o
    if with_stats:
        @pl.when(i == 0)
        def _():
            so_ref[...] = jnp.zeros_like(so_ref)

        so_ref[0:1, :] += jnp.sum(o, axis=0, keepdims=True)
        so_ref[1:2, :] += jnp.sum(o * o, axis=0, keepdims=True)


def _head4(u7, st7, bg, bb, w8t):
    body = functools.partial(_mlp_body, with_stats=True)
    return pl.pallas_call(
        body,
        grid=(B * FB,),
        in_specs=[
            pl.BlockSpec((NB, 512), lambda i: (i, 0)),
            pl.BlockSpec((8, 512), lambda i: (0, 0)),
            pl.BlockSpec((1, 512), lambda i: (0, 0)),
            pl.BlockSpec((1, 512), lambda i: (0, 0)),
            pl.BlockSpec((512, 256), lambda i: (0, 0)),
        ],
        out_specs=[
            pl.BlockSpec((NB, 256), lambda i: (i, 0)),
            pl.BlockSpec((8, 256), lambda i: (0, 0)),
        ],
        out_shape=[
            jax.ShapeDtypeStruct((B * N, 256), F32),
            jax.ShapeDtypeStruct((8, 256), F32),
        ],
    )(u7, st7, bg, bb, w8t)


def _head5_body(u_ref, st_ref, bg_ref, bb_ref, w_ref, o_ref):
    scale, shift = _affine(st_ref, bg_ref, bb_ref, CNT_PT)
    r = _lrelu(u_ref[...] * scale + shift)
    o_ref[...] = _bdot(r, w_ref)


def _head5(u8, st8, bg, bb, w9t):
    return pl.pallas_call(
        _head5_body,
        grid=(B * FB,),
        in_specs=[
            pl.BlockSpec((NB, 256), lambda i: (i, 0)),
            pl.BlockSpec((8, 256), lambda i: (0, 0)),
            pl.BlockSpec((1, 256), lambda i: (0, 0)),
            pl.BlockSpec((1, 256), lambda i: (0, 0)),
            pl.BlockSpec((256, 16), lambda i: (0, 0)),
        ],
        out_specs=pl.BlockSpec((NB, 16), lambda i: (i, 0)),
        out_shape=jax.ShapeDtypeStruct((B * N, 16), F32),
    )(u8, st8, bg, bb, w9t)


def kernel(x, params):
    p = params
    bf = lambda w: w.astype(BF16)
    w1t = bf(p['W1'].T)
    w2t = bf(p['W2'].T)
    w3t = bf(p['W3'].T)
    w4t = bf(p['W4'].T)
    w5t = bf(p['W5'].T)
    w6 = p['W6']
    w6a, w6b, w6c = bf(w6[:, :64].T), bf(w6[:, 64:128].T), bf(w6[:, 128:].T)
    w7 = p['W7']
    w7g = bf(w7[:, :1024].T)
    w7z1 = bf(w7[:, 1024:1088].T)
    w7z2 = bf(w7[:, 1088:1152].T)
    w7z3 = bf(w7[:, 1152:].T)
    w8t = bf(p['W8'].T)
    w9t = bf(jnp.pad(p['W9'], ((0, 3), (0, 0))).T)

    def bn(name):
        return p[name + '_g'].reshape(1, -1), p[name + '_b'].reshape(1, -1)

    g1g, g1b = bn('bn1')
    g2g, g2b = bn('bn2')
    g3g, g3b = bn('bn3')
    g4g, g4b = bn('bn4')
    g5g, g5b = bn('bn5')
    g6g, g6b = bn('bn6')
    g7g, g7b = bn('bn7')
    g8g, g8b = bn('bn8')

    x_rows = x.transpose(0, 2, 1).reshape(B * N, C0)
    x_pad = jnp.pad(x_rows, ((0, 0), (0, 128 - C0)))
    idx1 = _knn(x_rows, x)
    g1 = _sc_gather(x_pad, _prep_idx(idx1)).reshape(B * K, N, 128)
    g1f = g1.reshape(B * K * N, 128)
    ss1 = _tpass('sum', g1f, x_rows, w1t)
    sv1 = _tpass('var', g1f, x_rows, w1t, ss1)
    m1, u1, ss2 = _convmax(g1, x_rows, w1t, ss1, sv1, g1g, g1b, w2t)
    sv2 = _uvar(u1.reshape(B * K * N, 64), ss2)
    x1, x1p = _fin(m1, ss2, sv2, g2g, g2b)

    x1c = x1.reshape(B, N, 64).transpose(0, 2, 1)
    idx2 = _knn(x1, x1c)
    g2 = _sc_gather(x1p, _prep_idx(idx2)).reshape(B * K, N, 128)
    g2f = g2.reshape(B * K * N, 128)
    ss3 = _tpass('sum', g2f, x1, w3t)
    sv3 = _tpass('var', g2f, x1, w3t, ss3)
    m2, u2, ss4 = _convmax(g2, x1, w3t, ss3, sv3, g3g, g3b, w4t)
    sv4 = _uvar(u2.reshape(B * K * N, 64), ss4)
    x2, x2p = _fin(m2, ss4, sv4, g4g, g4b)

    x2c = x2.reshape(B, N, 64).transpose(0, 2, 1)
    idx3 = _knn(x2, x2c)
    g3 = _sc_gather(x2p, _prep_idx(idx3)).reshape(B * K, N, 128)
    m3, st5 = _conv5(g3, x2, w5t)
    x3 = _fin3(m3, st5, g5g, g5b)

    st6, gmax = _head1(x1, x2, x3, w6a, w6b, w6c)
    c7 = _head2(gmax, st6, g6g, g6b, w7g)
    u7, st7 = _head3(x1, x2, x3, w7z1, w7z2, w7z3, c7)
    u8, st8 = _head4(u7, st7, g7g, g7b, w8t)
    y = _head5(u8, st8, g8g, g8b, w9t)
    return y[:, :13].reshape(B, N, 13).transpose(0, 2, 1)
